# baseline (device time: 21668 ns/iter reference)
import os

import jax
import jax.numpy as jnp
from jax import lax
from jax.experimental import pallas as pl
from jax.experimental.pallas import tpu as pltpu

_PHASES = int(os.environ.get("DSM_PHASES", "3"))

N_DEV = 32
R0, R1 = 16, 128
H = R0 // 2
MH = H * R1
NCHUNK = 4


def kernel(x):
    m, n = x.shape
    assert m == R0 * R1, (m, n)

    def body(x_ref, out_ref, acc_ref, send_sems, recv_sems):
        p = lax.axis_index("i")
        z = p // 8
        r8 = p % 8
        y = r8 // 2
        xr = p % 2

        x_partner = p + 1 - 2 * xr
        y_targets = []
        for d in (1, 2, 3):
            yp = (y + d) % 4
            par = (y + yp) % 2
            xr_t = xr + par - 2 * xr * par
            y_targets.append((z * 8 + yp * 2 + xr_t, 5 - d, d))
        z_targets = []
        for d in (1, 2, 3):
            zp = (z + d) % 4
            z_targets.append((zp * 8 + r8, 8 - d, 3 + d))

        barrier = pltpu.get_barrier_semaphore()
        for pos in [x_partner] + [t[0] for t in y_targets] + [t[0] for t in z_targets]:
            pl.semaphore_signal(
                barrier, inc=1, device_id=(pos,),
                device_id_type=pl.DeviceIdType.MESH,
            )

        def local_pass(h):
            x3 = x_ref[h * MH:(h + 1) * MH, :].reshape(H, R1, n)
            nc = n // NCHUNK
            rm = rs = None
            chunk_m = []
            for c in range(NCHUNK):
                xc = x3[:, :, c * nc:(c + 1) * nc]
                cmax = jnp.max(xc, axis=2)
                if rm is None:
                    m_new = cmax
                else:
                    m_new = jnp.maximum(rm, cmax)
                ec = jnp.exp(xc - m_new[:, :, None])
                csum = jnp.sum(ec, axis=2)
                if rs is None:
                    rs = csum
                else:
                    rs = rs * jnp.exp(rm - m_new) + csum
                rm = m_new
                chunk_m.append(m_new)
                out_ref[h * MH:(h + 1) * MH, c * nc:(c + 1) * nc] = (
                    ec.reshape(MH, nc).astype(jnp.bfloat16)
                )
            acc_ref[h, 0, 0] = rm
            acc_ref[h, 0, 1] = rs
            return chunk_m

        def start_phase(h, targets):
            rdmas = []
            for pos, dst_slot, sem_i in targets:
                r = pltpu.make_async_remote_copy(
                    src_ref=acc_ref.at[h, 0],
                    dst_ref=acc_ref.at[h, dst_slot],
                    send_sem=send_sems.at[h * 7 + sem_i],
                    recv_sem=recv_sems.at[h * 7 + sem_i],
                    device_id=(pos,),
                    device_id_type=pl.DeviceIdType.MESH,
                )
                r.start()
                rdmas.append(r)
            return rdmas

        def finish_phase(h, rdmas, slots):
            for r in rdmas:
                r.wait()
            ms = [acc_ref[h, i, 0] for i in slots]
            ss = [acc_ref[h, i, 1] for i in slots]
            gm = ms[0]
            for mi in ms[1:]:
                gm = jnp.maximum(gm, mi)
            gs = ss[0] * jnp.exp(ms[0] - gm)
            for mi, si in zip(ms[1:], ss[1:]):
                gs = gs + si * jnp.exp(mi - gm)
            acc_ref[h, 0, 0] = gm
            acc_ref[h, 0, 1] = gs
            return gm, gs

        def rescale(h, chunk_m, gm, gs):
            nc = n // NCHUNK
            inv = 1.0 / gs
            for c in range(NCHUNK):
                scale = (jnp.exp(chunk_m[c] - gm) * inv).astype(jnp.bfloat16)
                e3 = out_ref[h * MH:(h + 1) * MH, c * nc:(c + 1) * nc].reshape(
                    H, R1, nc
                )
                out_ref[h * MH:(h + 1) * MH, c * nc:(c + 1) * nc] = (
                    (e3 * scale[:, :, None]).reshape(MH, nc)
                )

        x_tgt = [(x_partner, 1, 0)]

        lmax_a = local_pass(0)
        pl.semaphore_wait(barrier, 7)
        gm_a = gs_a = gm_b = gs_b = None
        if _PHASES >= 1:
            p1a = start_phase(0, x_tgt)
        lmax_b = local_pass(1)
        if _PHASES >= 1:
            p1b = start_phase(1, x_tgt)
            gm_a, gs_a = finish_phase(0, p1a, [0, 1])
        if _PHASES >= 2:
            p2a = start_phase(0, y_targets)
        if _PHASES >= 1:
            gm_b, gs_b = finish_phase(1, p1b, [0, 1])
        if _PHASES >= 2:
            p2b = start_phase(1, y_targets)
            gm_a, gs_a = finish_phase(0, p2a, [0, 2, 3, 4])
        if _PHASES >= 3:
            p3a = start_phase(0, z_targets)
        if _PHASES >= 2:
            gm_b, gs_b = finish_phase(1, p2b, [0, 2, 3, 4])
        if _PHASES >= 3:
            p3b = start_phase(1, z_targets)
            gm_a, gs_a = finish_phase(0, p3a, [0, 5, 6, 7])
        if gm_a is None:
            gm_a, gs_a = acc_ref[0, 0, 0], acc_ref[0, 0, 1]
        rescale(0, lmax_a, gm_a, gs_a)
        if _PHASES >= 3:
            gm_b, gs_b = finish_phase(1, p3b, [0, 5, 6, 7])
        if gm_b is None:
            gm_b, gs_b = acc_ref[1, 0, 0], acc_ref[1, 0, 1]
        rescale(1, lmax_b, gm_b, gs_b)

    return pl.pallas_call(
        body,
        out_shape=jax.ShapeDtypeStruct((m, n), jnp.bfloat16),
        in_specs=[pl.BlockSpec(memory_space=pltpu.VMEM)],
        out_specs=pl.BlockSpec(memory_space=pltpu.VMEM),
        scratch_shapes=[
            pltpu.VMEM((2, 8, 2, H, R1), jnp.float32),
            pltpu.SemaphoreType.DMA((14,)),
            pltpu.SemaphoreType.DMA((14,)),
        ],
        compiler_params=pltpu.CompilerParams(collective_id=0),
    )(x)


# device time: 19961 ns/iter; 1.0855x vs baseline; 1.0855x over previous
import jax
import jax.numpy as jnp
from jax import lax
from jax.experimental import pallas as pl
from jax.experimental.pallas import tpu as pltpu

N_DEV = 32
R0, R1 = 16, 128
H = R0 // 2
MH = H * R1


def kernel(x):
    m, n = x.shape
    assert m == R0 * R1, (m, n)

    def body(x_ref, out_ref, acc_ref, send_sems, recv_sems):
        p = lax.axis_index("i")
        z = p // 8
        r8 = p % 8
        y = r8 // 2
        xr = p % 2

        x_partner = p + 1 - 2 * xr
        y_targets = []
        for d in (1, 2, 3):
            yp = (y + d) % 4
            par = (y + yp) % 2
            xr_t = xr + par - 2 * xr * par
            y_targets.append((z * 8 + yp * 2 + xr_t, 5 - d, d))
        z_targets = []
        for d in (1, 2, 3):
            zp = (z + d) % 4
            z_targets.append((zp * 8 + r8, 8 - d, 3 + d))
        x_tgt = [(x_partner, 1, 0)]

        barrier = pltpu.get_barrier_semaphore()
        for pos in [x_partner] + [t[0] for t in y_targets] + [t[0] for t in z_targets]:
            pl.semaphore_signal(
                barrier, inc=1, device_id=(pos,),
                device_id_type=pl.DeviceIdType.MESH,
            )

        def half_x(h):
            return x_ref[h * MH:(h + 1) * MH, :].reshape(H, R1, n)

        def stats_pass(h):
            x3 = half_x(h)
            lmax = jnp.max(x3, axis=2)
            lsum = jnp.sum(jnp.exp(x3 - lmax[:, :, None]), axis=2)
            acc_ref[h, 0, 0] = lmax
            acc_ref[h, 0, 1] = lsum
            return lmax

        def store_e(h, lmax):
            e = jnp.exp(half_x(h) - lmax[:, :, None])
            out_ref[h * MH:(h + 1) * MH, :] = (
                e.reshape(MH, n).astype(jnp.bfloat16)
            )

        def start_phase(h, targets):
            rdmas = []
            for pos, dst_slot, sem_i in targets:
                r = pltpu.make_async_remote_copy(
                    src_ref=acc_ref.at[h, 0],
                    dst_ref=acc_ref.at[h, dst_slot],
                    send_sem=send_sems.at[h * 7 + sem_i],
                    recv_sem=recv_sems.at[h * 7 + sem_i],
                    device_id=(pos,),
                    device_id_type=pl.DeviceIdType.MESH,
                )
                r.start()
                rdmas.append(r)
            return rdmas

        def finish_phase(h, rdmas, slots):
            for r in rdmas:
                r.wait()
            ms = [acc_ref[h, i, 0] for i in slots]
            ss = [acc_ref[h, i, 1] for i in slots]
            gm = ms[0]
            for mi in ms[1:]:
                gm = jnp.maximum(gm, mi)
            gs = ss[0] * jnp.exp(ms[0] - gm)
            for mi, si in zip(ms[1:], ss[1:]):
                gs = gs + si * jnp.exp(mi - gm)
            acc_ref[h, 0, 0] = gm
            acc_ref[h, 0, 1] = gs
            return gm, gs

        def rescale(h, lmax, gm, gs):
            scale = (jnp.exp(lmax - gm) / gs).astype(jnp.bfloat16)
            e3 = out_ref[h * MH:(h + 1) * MH, :].reshape(H, R1, n)
            out_ref[h * MH:(h + 1) * MH, :] = (
                (e3 * scale[:, :, None]).reshape(MH, n)
            )

        lmax_a = stats_pass(0)
        pl.semaphore_wait(barrier, 7)
        p1a = start_phase(0, x_tgt)
        lmax_b = stats_pass(1)
        p1b = start_phase(1, x_tgt)
        store_e(0, lmax_a)
        finish_phase(0, p1a, [0, 1])
        p2a = start_phase(0, y_targets)
        finish_phase(1, p1b, [0, 1])
        p2b = start_phase(1, y_targets)
        store_e(1, lmax_b)
        finish_phase(0, p2a, [0, 2, 3, 4])
        p3a = start_phase(0, z_targets)
        finish_phase(1, p2b, [0, 2, 3, 4])
        p3b = start_phase(1, z_targets)
        gm_a, gs_a = finish_phase(0, p3a, [0, 5, 6, 7])
        rescale(0, lmax_a, gm_a, gs_a)
        gm_b, gs_b = finish_phase(1, p3b, [0, 5, 6, 7])
        rescale(1, lmax_b, gm_b, gs_b)

    return pl.pallas_call(
        body,
        out_shape=jax.ShapeDtypeStruct((m, n), jnp.bfloat16),
        in_specs=[pl.BlockSpec(memory_space=pltpu.VMEM)],
        out_specs=pl.BlockSpec(memory_space=pltpu.VMEM),
        scratch_shapes=[
            pltpu.VMEM((2, 8, 2, H, R1), jnp.float32),
            pltpu.SemaphoreType.DMA((14,)),
            pltpu.SemaphoreType.DMA((14,)),
        ],
        compiler_params=pltpu.CompilerParams(collective_id=0),
    )(x)
